# baseline (speedup 1.0000x reference)
import jax
import jax.numpy as jnp
from jax import lax
from jax.experimental import pallas as pl
from jax.experimental.pallas import tpu as pltpu
from jax.experimental.pallas import tpu_sc as plsc

_B = 16384
_NW = 32
_CHUNK = _B // _NW
_G = 16
_NGRP = _CHUNK // _G
_GB = 32
_NGRPB = _CHUNK // _GB

_BIG10 = [(0, 0), (1, 1), (8, 8), (9, 9), (13, 13), (0, 14), (1, 15)]
_BIG1 = [(0, 16), (1, 17)]
_SMALL = [(2, 2, 9, 9), (6, 6, 4, 4), (7, 7, 7, 7)]
_CGJOBS = [(11, 11, 201, 10), (3, 3, 21, 10), (4, 4, 13, 10),
           (5, 5, 22, 10), (10, 10, 12, 10), (12, 12, 192, 10)]


def _sc_gather_all(idxs, tables):
    n_idx = len(idxs)
    mesh = plsc.VectorSubcoreMesh(core_axis_name="c", subcore_axis_name="s")
    out_type = [jax.ShapeDtypeStruct((10, _B), jnp.float32)
                for _ in range(18)]
    scratch_types = [pltpu.VMEM((_CHUNK,), jnp.int32) for _ in range(n_idx)]
    scratch_types += [
        pltpu.VMEM((_GB * 8, 10), jnp.float32),
        pltpu.VMEM((_GB * 8, 10), jnp.float32),
        pltpu.VMEM((_G * 8, 1), jnp.float32),
        pltpu.VMEM((10, _CHUNK), jnp.float32),
    ]
    small_slots = {}
    for (i, t, v, w) in _SMALL:
        small_slots[t] = len(scratch_types)
        scratch_types.append(pltpu.VMEM((v, w), jnp.float32))
    cg_slot = len(scratch_types)
    scratch_types.append(pltpu.VMEM((201, 10), jnp.float32))
    n_scalar_scratch = len(scratch_types)
    scratch_types += [pltpu.SemaphoreType.DMA] * 7

    cp = pltpu.CompilerParams(needs_layout_passes=False)

    @pl.kernel(out_type=out_type, mesh=mesh, scratch_types=scratch_types,
               compiler_params=cp)
    def body(*refs):
        idx_refs = refs[:n_idx]
        tbl_refs = refs[n_idx:n_idx + 18]
        out_refs = refs[n_idx + 18:n_idx + 36]
        scr = refs[n_idx + 36:]
        idx_v = scr[:n_idx]
        buf10 = (scr[n_idx], scr[n_idx + 1])
        buf1 = scr[n_idx + 2]
        packed = scr[n_idx + 3]
        smalls = {t: scr[s] for t, s in small_slots.items()}
        cgbuf = scr[cg_slot]
        (sem_i, sem_g0, sem_g1, sem_h0, sem_h1,
         sem_t, sem_w) = scr[n_scalar_scratch:]
        sem_g = (sem_g0, sem_g1)
        wid = lax.axis_index("s") * 2 + lax.axis_index("c")
        base = wid * _CHUNK
        iota16 = lax.broadcasted_iota(jnp.int32, (16,), 0)

        for i in range(n_idx):
            pltpu.async_copy(
                idx_refs[i].at[pl.ds(base, _CHUNK)], idx_v[i], sem_i)
        for t in small_slots:
            pltpu.async_copy(tbl_refs[t], smalls[t], sem_t)
        pltpu.async_copy(tbl_refs[_CGJOBS[0][1]], cgbuf, sem_t)
        for i in range(n_idx):
            pltpu.make_async_copy(
                idx_refs[i].at[pl.ds(base, _CHUNK)], idx_v[i], sem_i).wait()

        def issue_gathers(iv, tbl, g, buf, sem, G):
            for kk in range(G // 16):
                v = iv[pl.ds(pl.multiple_of(g * G + kk * 16, 16), 16)]
                for j in range(16):
                    t8 = pl.multiple_of(jnp.bitwise_and(v[j], -8), 8)
                    pltpu.make_async_copy(
                        tbl.at[pl.ds(t8, 8), :],
                        buf.at[pl.ds((kk * 16 + j) * 8, 8), :],
                        sem,
                    ).start()

        def drain_gathers(tbl, buf, sem, G):
            pltpu.make_async_copy(
                tbl.at[pl.ds(0, G * 8), :], buf, sem).wait()

        def select(iv, w, g, buf, G):
            for kk in range(G // 16):
                v = iv[pl.ds(pl.multiple_of(g * G + kk * 16, 16), 16)]
                rows = (iota16 + kk * 16) * 8 + jnp.bitwise_and(v, 7)
                for c in range(w):
                    col = plsc.load_gather(
                        buf, [rows, jnp.full((16,), c, jnp.int32)])
                    packed[c, pl.ds(
                        pl.multiple_of(g * G + kk * 16, 16), 16)] = col

        def wait_writeback(prev):
            if prev is not None:
                pout, pw = prev
                pltpu.make_async_copy(
                    packed, pout.at[:, pl.ds(base, _CHUNK)], sem_w).wait()

        def writeback(out, w):
            pltpu.make_async_copy(
                packed, out.at[:, pl.ds(base, _CHUNK)], sem_w).start()

        prev = None
        for (i, t) in _BIG10:
            iv = idx_v[i]
            tbl = tbl_refs[t]
            out = out_refs[t]
            issue_gathers(iv, tbl, 0, buf10[0], sem_g[0], _GB)
            wait_writeback(prev)

            @pl.loop(0, _NGRPB // 2)
            def _(h, iv=iv, tbl=tbl):
                g = h * 2
                issue_gathers(iv, tbl, g + 1, buf10[1], sem_g[1], _GB)
                drain_gathers(tbl, buf10[0], sem_g[0], _GB)
                select(iv, 10, g, buf10[0], _GB)

                @pl.when(g + 2 < _NGRPB)
                def _():
                    issue_gathers(iv, tbl, g + 2, buf10[0], sem_g[0], _GB)
                drain_gathers(tbl, buf10[1], sem_g[1], _GB)
                select(iv, 10, g + 1, buf10[1], _GB)

            writeback(out, 10)
            prev = (out, 10)

        for (i, t) in _BIG1:
            iv = idx_v[i]
            tbl = tbl_refs[t]
            out = out_refs[t]
            wait_writeback(prev)

            @pl.loop(0, _NGRP)
            def _(g, iv=iv, tbl=tbl):
                issue_gathers(iv, tbl, g, buf1, sem_h0, _G)
                drain_gathers(tbl, buf1, sem_h0, _G)
                select(iv, 1, g, buf1, _G)

            writeback(out, 1)
            prev = (out, 1)

        for t in small_slots:
            pltpu.make_async_copy(tbl_refs[t], smalls[t], sem_t).wait()

        def small_job(i, t, w, buf, prev):
            iv = idx_v[i]
            out = out_refs[t]
            wait_writeback(prev)

            @pl.loop(0, _NGRP)
            def _(g, iv=iv, w=w, buf=buf):
                v = iv[pl.ds(pl.multiple_of(g * _G, _G), _G)]
                for c in range(w):
                    col = plsc.load_gather(
                        buf, [v, jnp.full((16,), c, jnp.int32)])
                    packed[c, pl.ds(pl.multiple_of(g * _G, _G), _G)] = col

            writeback(out, w)
            return (out, w)

        for (i, t, v_, w) in _SMALL:
            prev = small_job(i, t, w, smalls[t], prev)

        for n, (i, t, v_, w) in enumerate(_CGJOBS):
            dst = cgbuf if v_ == 201 else cgbuf.at[pl.ds(0, v_), :]
            pltpu.make_async_copy(tbl_refs[t], dst, sem_t).wait()
            prev = small_job(i, t, w, dst, prev)
            if n + 1 < len(_CGJOBS):
                (_ni, nt, nv, _nw) = _CGJOBS[n + 1]
                pltpu.async_copy(
                    tbl_refs[nt], cgbuf.at[pl.ds(0, nv), :], sem_t)
        wait_writeback(prev)

    return body(*idxs, *tables)


def _mlp_body(cat_ref, mfm_ref, mfs_ref, bm_ref, bs_ref, w1_ref, b1_ref,
              w2_ref, b2_ref, w3_ref, b3_ref, w4m_ref, w4h_ref, b4_ref,
              out_ref):
    x = cat_ref[...]
    h1 = jnp.maximum(
        jnp.dot(w1_ref[...], x, preferred_element_type=jnp.float32)
        + b1_ref[...], 0.0)
    h2 = jnp.maximum(
        jnp.dot(w2_ref[...], h1, preferred_element_type=jnp.float32)
        + b2_ref[...], 0.0)
    h3 = jnp.maximum(
        jnp.dot(w3_ref[...], h2, preferred_element_type=jnp.float32)
        + b3_ref[...], 0.0)
    mf = mfm_ref[...] * mfs_ref[...]
    logit = (jnp.dot(w4m_ref[...], mf, preferred_element_type=jnp.float32)
             + jnp.dot(w4h_ref[...], h3, preferred_element_type=jnp.float32)
             + b4_ref[...] + bm_ref[...] + bs_ref[...])
    out_ref[...] = jax.nn.sigmoid(logit)


def _mlp_call(cat, mfm, mfs, bm, bs, w1, b1, w2, b2, w3, b3, w4m, w4h, b4):
    T = 2048
    grid = (_B // T,)
    cdim = cat.shape[0]
    h1d = w1.shape[0]

    def col_spec(d):
        return pl.BlockSpec((d, T), lambda i: (0, i))

    def full_spec(a, b):
        return pl.BlockSpec((a, b), lambda i: (0, 0))

    return pl.pallas_call(
        _mlp_body,
        grid=grid,
        in_specs=[
            col_spec(cdim), col_spec(10), col_spec(10), col_spec(1),
            col_spec(1),
            full_spec(h1d, cdim), full_spec(h1d, 1),
            full_spec(cdim, h1d), full_spec(cdim, 1),
            full_spec(10, cdim), full_spec(10, 1),
            full_spec(1, 10), full_spec(1, 10), full_spec(1, 1),
        ],
        out_specs=col_spec(1),
        out_shape=jax.ShapeDtypeStruct((1, _B), jnp.float32),
    )(cat, mfm, mfs, bm, bs, w1, b1, w2, b2, w3, b3, w4m, w4h, b4)


def kernel(msno, song_id, source_system_tab, source_screen_name, source_type,
           city, gender, registered_via, composer, lyricist, language,
           country, genre, artist, msno_nn_w, msno_mf_w, msno_bias_w,
           song_id_nn_w, song_id_mf_w, song_id_bias_w, source_system_tab_w,
           source_screen_name_w, source_type_w, city_w, gender_w,
           registered_via_w, composer_w, lyricist_w, language_w, country_w,
           genre_w, artist_w, W1, b1, W2, b2, W3, b3, W4, b4):
    idxs = [msno, song_id, source_system_tab, source_screen_name, source_type,
            city, gender, registered_via, composer, lyricist, language,
            country, genre, artist]
    tables = [msno_nn_w, song_id_nn_w, source_system_tab_w,
              source_screen_name_w, source_type_w, city_w, gender_w,
              registered_via_w, composer_w, lyricist_w, language_w,
              country_w, genre_w, artist_w,
              msno_mf_w, song_id_mf_w, msno_bias_w, song_id_bias_w]
    g = _sc_gather_all(idxs, tables)
    widths = [10, 10, 9, 10, 10, 10, 4, 7, 10, 10, 10, 10, 10, 10]
    catT = jnp.concatenate(
        [g[f][:w] for f, w in enumerate(widths)], axis=0)
    mfmT, mfsT = g[14], g[15]
    bmT, bsT = g[16][:1], g[17][:1]
    outT = _mlp_call(
        catT, mfmT, mfsT, bmT, bsT,
        W1, b1.reshape(-1, 1), W2, b2.reshape(-1, 1),
        W3, b3.reshape(-1, 1),
        W4[:, :10], W4[:, 10:], b4.reshape(1, 1))
    return outT.reshape(_B, 1)


# speedup vs baseline: 1.0452x; 1.0452x over previous
import jax
import jax.numpy as jnp
from jax import lax
from jax.experimental import pallas as pl
from jax.experimental.pallas import tpu as pltpu
from jax.experimental.pallas import tpu_sc as plsc

_B = 16384
_NW = 32
_CHUNK = _B // _NW
_G = 16
_NGRP = _CHUNK // _G
_GB = 32
_NGRPB = _CHUNK // _GB

_BIG10 = [(1, 1), (8, 8), (9, 9), (13, 13), (1, 15)]
_BIG1 = [(1, 17)]
_SMALL = [(2, 2, 9, 9), (6, 6, 4, 4), (7, 7, 7, 7)]
_CGJOBS = [(11, 11, 201, 10), (3, 3, 21, 10), (4, 4, 13, 10),
           (5, 5, 22, 10), (10, 10, 12, 10), (12, 12, 192, 10)]


def _sc_gather_all(idxs, tables):
    n_idx = len(idxs)
    mesh = plsc.VectorSubcoreMesh(core_axis_name="c", subcore_axis_name="s")
    out_type = [jax.ShapeDtypeStruct((10, _B), jnp.float32)
                for _ in range(18)]
    scratch_types = [pltpu.VMEM((_CHUNK,), jnp.int32) for _ in range(n_idx)]
    scratch_types += [
        pltpu.VMEM((_GB * 8, 10), jnp.float32),
        pltpu.VMEM((_GB * 8, 10), jnp.float32),
        pltpu.VMEM((_G * 8, 1), jnp.float32),
        pltpu.VMEM((10, _CHUNK), jnp.float32),
    ]
    small_slots = {}
    for (i, t, v, w) in _SMALL:
        small_slots[t] = len(scratch_types)
        scratch_types.append(pltpu.VMEM((v, w), jnp.float32))
    cg_slot = len(scratch_types)
    scratch_types.append(pltpu.VMEM((201, 10), jnp.float32))
    n_scalar_scratch = len(scratch_types)
    scratch_types += [pltpu.SemaphoreType.DMA] * 7

    cp = pltpu.CompilerParams(needs_layout_passes=False)

    @pl.kernel(out_type=out_type, mesh=mesh, scratch_types=scratch_types,
               compiler_params=cp)
    def body(*refs):
        idx_refs = refs[:n_idx]
        tbl_refs = refs[n_idx:n_idx + 18]
        out_refs = refs[n_idx + 18:n_idx + 36]
        scr = refs[n_idx + 36:]
        idx_v = scr[:n_idx]
        buf10 = (scr[n_idx], scr[n_idx + 1])
        buf1 = scr[n_idx + 2]
        packed = scr[n_idx + 3]
        smalls = {t: scr[s] for t, s in small_slots.items()}
        cgbuf = scr[cg_slot]
        (sem_i, sem_g0, sem_g1, sem_h0, sem_h1,
         sem_t, sem_w) = scr[n_scalar_scratch:]
        sem_g = (sem_g0, sem_g1)
        wid = lax.axis_index("s") * 2 + lax.axis_index("c")
        base = wid * _CHUNK
        iota16 = lax.broadcasted_iota(jnp.int32, (16,), 0)

        for i in range(n_idx):
            pltpu.async_copy(
                idx_refs[i].at[pl.ds(base, _CHUNK)], idx_v[i], sem_i)
        for t in small_slots:
            pltpu.async_copy(tbl_refs[t], smalls[t], sem_t)
        pltpu.async_copy(tbl_refs[_CGJOBS[0][1]], cgbuf, sem_t)
        for i in range(n_idx):
            pltpu.make_async_copy(
                idx_refs[i].at[pl.ds(base, _CHUNK)], idx_v[i], sem_i).wait()

        def issue_gathers(iv, tbl, g, buf, sem, G):
            for kk in range(G // 16):
                v = iv[pl.ds(pl.multiple_of(g * G + kk * 16, 16), 16)]
                for j in range(16):
                    t8 = pl.multiple_of(jnp.bitwise_and(v[j], -8), 8)
                    pltpu.make_async_copy(
                        tbl.at[pl.ds(t8, 8), :],
                        buf.at[pl.ds((kk * 16 + j) * 8, 8), :],
                        sem,
                    ).start()

        def drain_gathers(tbl, buf, sem, G):
            pltpu.make_async_copy(
                tbl.at[pl.ds(0, G * 8), :], buf, sem).wait()

        def select(iv, w, g, buf, G):
            for kk in range(G // 16):
                v = iv[pl.ds(pl.multiple_of(g * G + kk * 16, 16), 16)]
                rows = (iota16 + kk * 16) * 8 + jnp.bitwise_and(v, 7)
                for c in range(w):
                    col = plsc.load_gather(
                        buf, [rows, jnp.full((16,), c, jnp.int32)])
                    packed[c, pl.ds(
                        pl.multiple_of(g * G + kk * 16, 16), 16)] = col

        def wait_writeback(prev):
            if prev is not None:
                pout, pw = prev
                pltpu.make_async_copy(
                    packed, pout.at[:, pl.ds(base, _CHUNK)], sem_w).wait()

        def writeback(out, w):
            pltpu.make_async_copy(
                packed, out.at[:, pl.ds(base, _CHUNK)], sem_w).start()

        prev = None
        for (i, t) in _BIG10:
            iv = idx_v[i]
            tbl = tbl_refs[t]
            out = out_refs[t]
            issue_gathers(iv, tbl, 0, buf10[0], sem_g[0], _GB)
            wait_writeback(prev)

            @pl.loop(0, _NGRPB // 2)
            def _(h, iv=iv, tbl=tbl):
                g = h * 2
                issue_gathers(iv, tbl, g + 1, buf10[1], sem_g[1], _GB)
                drain_gathers(tbl, buf10[0], sem_g[0], _GB)
                select(iv, 10, g, buf10[0], _GB)

                @pl.when(g + 2 < _NGRPB)
                def _():
                    issue_gathers(iv, tbl, g + 2, buf10[0], sem_g[0], _GB)
                drain_gathers(tbl, buf10[1], sem_g[1], _GB)
                select(iv, 10, g + 1, buf10[1], _GB)

            writeback(out, 10)
            prev = (out, 10)

        for (i, t) in _BIG1:
            iv = idx_v[i]
            tbl = tbl_refs[t]
            out = out_refs[t]
            wait_writeback(prev)

            @pl.loop(0, _NGRP)
            def _(g, iv=iv, tbl=tbl):
                issue_gathers(iv, tbl, g, buf1, sem_h0, _G)
                drain_gathers(tbl, buf1, sem_h0, _G)
                select(iv, 1, g, buf1, _G)

            writeback(out, 1)
            prev = (out, 1)

        for t in small_slots:
            pltpu.make_async_copy(tbl_refs[t], smalls[t], sem_t).wait()

        def small_job(i, t, w, buf, prev):
            iv = idx_v[i]
            out = out_refs[t]
            wait_writeback(prev)

            @pl.loop(0, _NGRP)
            def _(g, iv=iv, w=w, buf=buf):
                v = iv[pl.ds(pl.multiple_of(g * _G, _G), _G)]
                for c in range(w):
                    col = plsc.load_gather(
                        buf, [v, jnp.full((16,), c, jnp.int32)])
                    packed[c, pl.ds(pl.multiple_of(g * _G, _G), _G)] = col

            writeback(out, w)
            return (out, w)

        for (i, t, v_, w) in _SMALL:
            prev = small_job(i, t, w, smalls[t], prev)

        for n, (i, t, v_, w) in enumerate(_CGJOBS):
            dst = cgbuf if v_ == 201 else cgbuf.at[pl.ds(0, v_), :]
            pltpu.make_async_copy(tbl_refs[t], dst, sem_t).wait()
            prev = small_job(i, t, w, dst, prev)
            if n + 1 < len(_CGJOBS):
                (_ni, nt, nv, _nw) = _CGJOBS[n + 1]
                pltpu.async_copy(
                    tbl_refs[nt], cgbuf.at[pl.ds(0, nv), :], sem_t)
        wait_writeback(prev)

    return body(*idxs, *tables)


def _tc_gather_body(idx_ref, nn_ref, mf_ref, b_ref, o1_ref, o2_ref, o3_ref,
                    sem):
    T = o1_ref.shape[0]

    def loop(r, carry):
        ix = idx_ref[r]
        pltpu.make_async_copy(
            nn_ref.at[pl.ds(ix, 1), :], o1_ref.at[pl.ds(r, 1), :], sem
        ).start()
        pltpu.make_async_copy(
            mf_ref.at[pl.ds(ix, 1), :], o2_ref.at[pl.ds(r, 1), :], sem
        ).start()
        pltpu.make_async_copy(
            b_ref.at[pl.ds(ix, 1), :], o3_ref.at[pl.ds(r, 1), :], sem
        ).start()
        return carry

    lax.fori_loop(0, T, loop, 0)
    pltpu.make_async_copy(nn_ref.at[pl.ds(0, T), :], o1_ref, sem).wait()
    pltpu.make_async_copy(mf_ref.at[pl.ds(0, T), :], o2_ref, sem).wait()
    pltpu.make_async_copy(b_ref.at[pl.ds(0, T), :], o3_ref, sem).wait()


def _tc_gather(idx, nn_w, mf_w, b_w):
    T = 2048
    grid = (_B // T,)
    return pl.pallas_call(
        _tc_gather_body,
        grid=grid,
        in_specs=[
            pl.BlockSpec((T,), lambda i: (i,), memory_space=pltpu.SMEM),
            pl.BlockSpec(memory_space=pltpu.MemorySpace.HBM),
            pl.BlockSpec(memory_space=pltpu.MemorySpace.HBM),
            pl.BlockSpec(memory_space=pltpu.MemorySpace.HBM),
        ],
        out_specs=[
            pl.BlockSpec((T, 10), lambda i: (i, 0)),
            pl.BlockSpec((T, 10), lambda i: (i, 0)),
            pl.BlockSpec((T, 1), lambda i: (i, 0)),
        ],
        out_shape=[
            jax.ShapeDtypeStruct((_B, 10), jnp.float32),
            jax.ShapeDtypeStruct((_B, 10), jnp.float32),
            jax.ShapeDtypeStruct((_B, 1), jnp.float32),
        ],
        scratch_shapes=[pltpu.SemaphoreType.DMA],
    )(idx, nn_w, mf_w, b_w)


def _mlp_body(cat_ref, mfm_ref, mfs_ref, bm_ref, bs_ref, w1_ref, b1_ref,
              w2_ref, b2_ref, w3_ref, b3_ref, w4m_ref, w4h_ref, b4_ref,
              out_ref):
    x = cat_ref[...]
    h1 = jnp.maximum(
        jnp.dot(w1_ref[...], x, preferred_element_type=jnp.float32)
        + b1_ref[...], 0.0)
    h2 = jnp.maximum(
        jnp.dot(w2_ref[...], h1, preferred_element_type=jnp.float32)
        + b2_ref[...], 0.0)
    h3 = jnp.maximum(
        jnp.dot(w3_ref[...], h2, preferred_element_type=jnp.float32)
        + b3_ref[...], 0.0)
    mf = mfm_ref[...] * mfs_ref[...]
    logit = (jnp.dot(w4m_ref[...], mf, preferred_element_type=jnp.float32)
             + jnp.dot(w4h_ref[...], h3, preferred_element_type=jnp.float32)
             + b4_ref[...] + bm_ref[...] + bs_ref[...])
    out_ref[...] = jax.nn.sigmoid(logit)


def _mlp_call(cat, mfm, mfs, bm, bs, w1, b1, w2, b2, w3, b3, w4m, w4h, b4):
    T = 2048
    grid = (_B // T,)
    cdim = cat.shape[0]
    h1d = w1.shape[0]

    def col_spec(d):
        return pl.BlockSpec((d, T), lambda i: (0, i))

    def full_spec(a, b):
        return pl.BlockSpec((a, b), lambda i: (0, 0))

    return pl.pallas_call(
        _mlp_body,
        grid=grid,
        in_specs=[
            col_spec(cdim), col_spec(10), col_spec(10), col_spec(1),
            col_spec(1),
            full_spec(h1d, cdim), full_spec(h1d, 1),
            full_spec(cdim, h1d), full_spec(cdim, 1),
            full_spec(10, cdim), full_spec(10, 1),
            full_spec(1, 10), full_spec(1, 10), full_spec(1, 1),
        ],
        out_specs=col_spec(1),
        out_shape=jax.ShapeDtypeStruct((1, _B), jnp.float32),
    )(cat, mfm, mfs, bm, bs, w1, b1, w2, b2, w3, b3, w4m, w4h, b4)


def kernel(msno, song_id, source_system_tab, source_screen_name, source_type,
           city, gender, registered_via, composer, lyricist, language,
           country, genre, artist, msno_nn_w, msno_mf_w, msno_bias_w,
           song_id_nn_w, song_id_mf_w, song_id_bias_w, source_system_tab_w,
           source_screen_name_w, source_type_w, city_w, gender_w,
           registered_via_w, composer_w, lyricist_w, language_w, country_w,
           genre_w, artist_w, W1, b1, W2, b2, W3, b3, W4, b4):
    idxs = [msno, song_id, source_system_tab, source_screen_name, source_type,
            city, gender, registered_via, composer, lyricist, language,
            country, genre, artist]
    tables = [msno_nn_w, song_id_nn_w, source_system_tab_w,
              source_screen_name_w, source_type_w, city_w, gender_w,
              registered_via_w, composer_w, lyricist_w, language_w,
              country_w, genre_w, artist_w,
              msno_mf_w, song_id_mf_w, msno_bias_w, song_id_bias_w]
    g = _sc_gather_all(idxs, tables)
    tc_nn, tc_mf, tc_b = _tc_gather(msno, msno_nn_w, msno_mf_w, msno_bias_w)
    widths = [10, 10, 9, 10, 10, 10, 4, 7, 10, 10, 10, 10, 10, 10]
    catT = jnp.concatenate(
        [tc_nn.T] + [g[f][:w] for f, w in list(enumerate(widths))[1:]],
        axis=0)
    mfmT, mfsT = tc_mf.T, g[15]
    bmT, bsT = tc_b.T, g[17][:1]
    outT = _mlp_call(
        catT, mfmT, mfsT, bmT, bsT,
        W1, b1.reshape(-1, 1), W2, b2.reshape(-1, 1),
        W3, b3.reshape(-1, 1),
        W4[:, :10], W4[:, 10:], b4.reshape(1, 1))
    return outT.reshape(_B, 1)
